# baseline (speedup 1.0000x reference)
import jax
import jax.numpy as jnp
from jax.experimental import pallas as pl
from jax.experimental.pallas import tpu as pltpu

_BF16 = jnp.bfloat16
_F32 = jnp.float32


def _pack_values(w1a, w1b, b1, w2a, w2b, b2, wf1):
    s1 = [jnp.pad(w1a, ((0, 0), (0, 0), (0, 8))),
          jnp.pad(w1b, ((0, 0), (0, 0), (0, 8)))]
    w1 = jnp.concatenate(
        [jnp.pad(s, ((j, 3 - j), (0, 0), (0, 0))).reshape(224, 128)
         for j in range(4) for s in s1], axis=1)
    s2 = [jnp.pad(w2a, ((0, 0), (0, 8), (0, 48))),
          jnp.pad(w2b, ((0, 0), (0, 8), (0, 48)))]
    w2 = jnp.concatenate(
        [jnp.pad(s, ((j, 1 - j), (0, 0), (0, 0))).reshape(768, 128)
         for j in range(2) for s in s2], axis=1)
    wf1p = jnp.pad(wf1, ((0, 0), (0, 48), (0, 0))).reshape(512, 128)
    b1t = jnp.tile(jnp.pad(b1, ((0, 0), (0, 8))), (1, 12))
    b2t = jnp.tile(jnp.pad(b2, ((0, 0), (0, 48))), (1, 4))
    return w1.astype(_BF16), b1t, w2.astype(_BF16), b2t, wf1p.astype(_BF16)


def _pack_body(w1a_ref, w1b_ref, b1_ref, w2a_ref, w2b_ref, b2_ref, wf1_ref,
               wf2_ref, w1_o, b1_o, w2_o, b2_o, wf1_o, wf2_o):
    w1, b1t, w2, b2t, wf1p = _pack_values(
        w1a_ref[...], w1b_ref[...], b1_ref[...], w2a_ref[...], w2b_ref[...],
        b2_ref[...], wf1_ref[...])
    w1_o[...] = w1
    b1_o[...] = b1t
    w2_o[...] = w2
    b2_o[...] = b2t
    wf1_o[...] = wf1p
    wf2_o[...] = wf2_ref[...].astype(_BF16)


def _net_body(x_ref, w1_ref, b1_ref, w2_ref, b2_ref, wf1_ref, bf1_ref,
              wf2_ref, bf2_ref, feat_ref, out_ref):
    tb = x_ref.shape[0]
    xv = x_ref[...].reshape(tb, 784)

    hp_chunks = []
    for g in range(6):
        lhs = xv[:, 112 * g:112 * g + 224].astype(_BF16)
        z = jnp.dot(lhs, w1_ref[...], preferred_element_type=_F32)
        for j2 in range(2):
            s = 512 * j2
            hp_chunks.append(jnp.maximum(
                jnp.maximum(z[:, s:s + 128], z[:, s + 128:s + 256]),
                jnp.maximum(z[:, s + 256:s + 384], z[:, s + 384:s + 512])))
    p1 = jnp.concatenate(hp_chunks, axis=1)
    p1 = jnp.maximum(p1 + b1_ref[...], 0.0).astype(_BF16)

    hp2_chunks = []
    for g in range(4):
        z = jnp.dot(p1[:, 256 * g:256 * g + 768], w2_ref[...],
                    preferred_element_type=_F32)
        hp2_chunks.append(jnp.maximum(
            jnp.maximum(z[:, 0:128], z[:, 128:256]),
            jnp.maximum(z[:, 256:384], z[:, 384:512])))
    p2 = jnp.concatenate(hp2_chunks, axis=1)
    p2 = jnp.maximum(p2 + b2_ref[...], 0.0).astype(_BF16)

    feat = jnp.maximum(
        jnp.dot(p2, wf1_ref[...], preferred_element_type=_F32) + bf1_ref[...],
        0.0)
    feat_ref[...] = feat
    out = (jnp.dot(feat.astype(_BF16), wf2_ref[...],
                   preferred_element_type=_F32) + bf2_ref[...])
    out_ref[...] = out[:, :10]


def kernel(x, w1a, w1b, b1, w2a, w2b, b2, wf1, bf1, wf2, bf2):
    b = x.shape[0]
    xs = x
    if b <= 512:
        tb, pad = b, 0
    else:
        tb = 512
        pad = (-b) % tb
    if pad:
        xs = jnp.pad(xs, ((0, pad), (0, 0), (0, 0), (0, 0)))

    packed = pl.pallas_call(
        _pack_body,
        out_shape=(jax.ShapeDtypeStruct((224, 1024), _BF16),
                   jax.ShapeDtypeStruct((1, 1536), _F32),
                   jax.ShapeDtypeStruct((768, 512), _BF16),
                   jax.ShapeDtypeStruct((1, 512), _F32),
                   jax.ShapeDtypeStruct((512, 128), _BF16),
                   jax.ShapeDtypeStruct((128, 128), _BF16)),
    )(w1a, w1b, b1, w2a, w2b, b2, wf1, wf2)
    w1p, b1t, w2p, b2t, wf1p, wf2p = packed

    bp = xs.shape[0]
    c2 = lambda i: (0, 0)
    feat, out = pl.pallas_call(
        _net_body,
        out_shape=(jax.ShapeDtypeStruct((bp, 128), _F32),
                   jax.ShapeDtypeStruct((bp, 10), _F32)),
        grid=(bp // tb,),
        in_specs=[
            pl.BlockSpec((tb, 1, 28, 28), lambda i: (i, 0, 0, 0)),
            pl.BlockSpec((224, 1024), c2),
            pl.BlockSpec((1, 1536), c2),
            pl.BlockSpec((768, 512), c2),
            pl.BlockSpec((1, 512), c2),
            pl.BlockSpec((512, 128), c2),
            pl.BlockSpec((1, 128), c2),
            pl.BlockSpec((128, 128), c2),
            pl.BlockSpec((1, 128), c2),
        ],
        out_specs=[
            pl.BlockSpec((tb, 128), lambda i: (i, 0)),
            pl.BlockSpec((tb, 10), lambda i: (i, 0)),
        ],
        compiler_params=pltpu.CompilerParams(
            dimension_semantics=("parallel",),
            vmem_limit_bytes=64 * 1024 * 1024,
        ),
    )(xs, w1p, b1t, w2p, b2t, wf1p, bf1, wf2p, bf2)
    if pad:
        feat, out = feat[:b], out[:b]
    return feat, out


# speedup vs baseline: 1.6094x; 1.6094x over previous
import jax
import jax.numpy as jnp
from jax.experimental import pallas as pl
from jax.experimental.pallas import tpu as pltpu

_BF16 = jnp.bfloat16
_F32 = jnp.float32


def _pack_values(w1a, w1b, b1, w2a, w2b, b2, wf1):
    s1 = [jnp.pad(w1a, ((0, 0), (0, 0), (0, 8))),
          jnp.pad(w1b, ((0, 0), (0, 0), (0, 8)))]
    w1 = jnp.concatenate(
        [jnp.pad(s, ((j, 3 - j), (0, 0), (0, 0))).reshape(224, 128)
         for j in range(4) for s in s1], axis=1)
    s2 = [jnp.pad(w2a, ((0, 0), (0, 8), (0, 48))),
          jnp.pad(w2b, ((0, 0), (0, 8), (0, 48)))]
    w2 = jnp.concatenate(
        [jnp.pad(s, ((j, 1 - j), (0, 0), (0, 0))).reshape(768, 128)
         for j in range(2) for s in s2], axis=1)
    wf1p = jnp.pad(wf1, ((0, 0), (0, 48), (0, 0))).reshape(512, 128)
    b1t = jnp.tile(jnp.pad(b1, ((0, 0), (0, 8))), (1, 12))
    b2t = jnp.tile(jnp.pad(b2, ((0, 0), (0, 48))), (1, 4))
    return w1.astype(_BF16), b1t, w2.astype(_BF16), b2t, wf1p.astype(_BF16)


def _pack_body(w1a_ref, w1b_ref, b1_ref, w2a_ref, w2b_ref, b2_ref, wf1_ref,
               wf2_ref, w1_o, b1_o, w2_o, b2_o, wf1_o, wf2_o):
    w1, b1t, w2, b2t, wf1p = _pack_values(
        w1a_ref[...], w1b_ref[...], b1_ref[...], w2a_ref[...], w2b_ref[...],
        b2_ref[...], wf1_ref[...])
    w1_o[...] = w1
    b1_o[...] = b1t
    w2_o[...] = w2
    b2_o[...] = b2t
    wf1_o[...] = wf1p
    wf2_o[...] = wf2_ref[...].astype(_BF16)


def _net_body(x_ref, w1_ref, b1_ref, w2_ref, b2_ref, wf1_ref, bf1_ref,
              wf2_ref, bf2_ref, feat_ref, out_ref):
    tb = x_ref.shape[0]
    xv = x_ref[...].reshape(tb, 784)

    hp_chunks = []
    for g in range(6):
        lhs = xv[:, 112 * g:112 * g + 224].astype(_BF16)
        z = jnp.dot(lhs, w1_ref[...], preferred_element_type=_F32)
        for j2 in range(2):
            s = 512 * j2
            hp_chunks.append(jnp.maximum(
                jnp.maximum(z[:, s:s + 128], z[:, s + 128:s + 256]),
                jnp.maximum(z[:, s + 256:s + 384], z[:, s + 384:s + 512])))
    p1 = jnp.concatenate(hp_chunks, axis=1)
    p1 = jnp.maximum(p1 + b1_ref[...], 0.0).astype(_BF16)

    hp2_chunks = []
    for g in range(4):
        z = jnp.dot(p1[:, 256 * g:256 * g + 768], w2_ref[...],
                    preferred_element_type=_F32)
        hp2_chunks.append(jnp.maximum(
            jnp.maximum(z[:, 0:128], z[:, 128:256]),
            jnp.maximum(z[:, 256:384], z[:, 384:512])))
    p2 = jnp.concatenate(hp2_chunks, axis=1)
    p2 = jnp.maximum(p2 + b2_ref[...], 0.0).astype(_BF16)

    feat = jnp.maximum(
        jnp.dot(p2, wf1_ref[...], preferred_element_type=_F32) + bf1_ref[...],
        0.0)
    feat_ref[...] = feat
    out = (jnp.dot(feat.astype(_BF16), wf2_ref[...],
                   preferred_element_type=_F32) + bf2_ref[...])
    out_ref[...] = out[:, :10]


def kernel(x, w1a, w1b, b1, w2a, w2b, b2, wf1, bf1, wf2, bf2):
    b = x.shape[0]
    xs = x.reshape(b, 28, 28)
    if b <= 1024:
        tb, pad = b, 0
    else:
        tb = 1024
        pad = (-b) % tb
    if pad:
        xs = jnp.pad(xs, ((0, pad), (0, 0), (0, 0)))

    packed = pl.pallas_call(
        _pack_body,
        out_shape=(jax.ShapeDtypeStruct((224, 1024), _BF16),
                   jax.ShapeDtypeStruct((1, 1536), _F32),
                   jax.ShapeDtypeStruct((768, 512), _BF16),
                   jax.ShapeDtypeStruct((1, 512), _F32),
                   jax.ShapeDtypeStruct((512, 128), _BF16),
                   jax.ShapeDtypeStruct((128, 128), _BF16)),
    )(w1a, w1b, b1, w2a, w2b, b2, wf1, wf2)
    w1p, b1t, w2p, b2t, wf1p, wf2p = packed

    bp = xs.shape[0]
    c2 = lambda i: (0, 0)
    feat, out = pl.pallas_call(
        _net_body,
        out_shape=(jax.ShapeDtypeStruct((bp, 128), _F32),
                   jax.ShapeDtypeStruct((bp, 10), _F32)),
        grid=(bp // tb,),
        in_specs=[
            pl.BlockSpec((tb, 28, 28), lambda i: (i, 0, 0)),
            pl.BlockSpec((224, 1024), c2),
            pl.BlockSpec((1, 1536), c2),
            pl.BlockSpec((768, 512), c2),
            pl.BlockSpec((1, 512), c2),
            pl.BlockSpec((512, 128), c2),
            pl.BlockSpec((1, 128), c2),
            pl.BlockSpec((128, 128), c2),
            pl.BlockSpec((1, 128), c2),
        ],
        out_specs=[
            pl.BlockSpec((tb, 128), lambda i: (i, 0)),
            pl.BlockSpec((tb, 10), lambda i: (i, 0)),
        ],
        compiler_params=pltpu.CompilerParams(
            dimension_semantics=("parallel",),
            vmem_limit_bytes=64 * 1024 * 1024,
        ),
    )(xs, w1p, b1t, w2p, b2t, wf1p, bf1, wf2p, bf2)
    if pad:
        feat, out = feat[:b], out[:b]
    return feat, out


# speedup vs baseline: 1.7835x; 1.1082x over previous
import jax
import jax.numpy as jnp
from jax.experimental import pallas as pl
from jax.experimental.pallas import tpu as pltpu

_BF16 = jnp.bfloat16
_F32 = jnp.float32


def _pack_values(w1a, w1b, b1, w2a, w2b, b2, wf1):
    s1 = [jnp.pad(w1a, ((0, 0), (0, 0), (0, 8))),
          jnp.pad(w1b, ((0, 0), (0, 0), (0, 8)))]
    w1 = jnp.concatenate(
        [jnp.pad(s, ((j, 3 - j), (0, 0), (0, 0))).reshape(224, 128)
         for j in range(4) for s in s1], axis=1)
    s2 = [jnp.pad(w2a, ((0, 0), (0, 8), (0, 48))),
          jnp.pad(w2b, ((0, 0), (0, 8), (0, 48)))]
    w2 = jnp.concatenate(
        [jnp.pad(s, ((j, 1 - j), (0, 0), (0, 0))).reshape(768, 128)
         for j in range(2) for s in s2], axis=1)
    wf1p = jnp.pad(wf1, ((0, 0), (0, 48), (0, 0))).reshape(512, 128)
    b1t = jnp.tile(jnp.pad(b1, ((0, 0), (0, 8))), (1, 12))
    b2t = jnp.tile(jnp.pad(b2, ((0, 0), (0, 48))), (1, 4))
    return w1.astype(_BF16), b1t, w2.astype(_BF16), b2t, wf1p.astype(_BF16)


def _pack_body(w1a_ref, w1b_ref, b1_ref, w2a_ref, w2b_ref, b2_ref, wf1_ref,
               wf2_ref, w1_o, b1_o, w2_o, b2_o, wf1_o, wf2_o):
    w1, b1t, w2, b2t, wf1p = _pack_values(
        w1a_ref[...], w1b_ref[...], b1_ref[...], w2a_ref[...], w2b_ref[...],
        b2_ref[...], wf1_ref[...])
    w1_o[...] = w1
    b1_o[...] = b1t
    w2_o[...] = w2
    b2_o[...] = b2t
    wf1_o[...] = wf1p
    wf2_o[...] = wf2_ref[...].astype(_BF16)


def _net_body(x_ref, w1_ref, b1_ref, w2_ref, b2_ref, wf1_ref, bf1_ref,
              wf2_ref, bf2_ref, feat_ref, out_ref):
    tb = x_ref.shape[0]
    xv = x_ref[...].astype(_BF16).reshape(tb, 784)

    hp_chunks = []
    for g in range(6):
        lhs = xv[:, 112 * g:112 * g + 224]
        z = jnp.dot(lhs, w1_ref[...], preferred_element_type=_F32)
        for j2 in range(2):
            s = 512 * j2
            hp_chunks.append(jnp.maximum(
                jnp.maximum(z[:, s:s + 128], z[:, s + 128:s + 256]),
                jnp.maximum(z[:, s + 256:s + 384], z[:, s + 384:s + 512])))
    p1 = jnp.concatenate(hp_chunks, axis=1)
    p1 = jnp.maximum(p1 + b1_ref[...], 0.0).astype(_BF16)

    hp2_chunks = []
    for g in range(4):
        z = jnp.dot(p1[:, 256 * g:256 * g + 768], w2_ref[...],
                    preferred_element_type=_F32)
        hp2_chunks.append(jnp.maximum(
            jnp.maximum(z[:, 0:128], z[:, 128:256]),
            jnp.maximum(z[:, 256:384], z[:, 384:512])))
    p2 = jnp.concatenate(hp2_chunks, axis=1)
    p2 = jnp.maximum(p2 + b2_ref[...], 0.0).astype(_BF16)

    feat = jnp.maximum(
        jnp.dot(p2, wf1_ref[...], preferred_element_type=_F32) + bf1_ref[...],
        0.0)
    feat_ref[...] = feat
    out = (jnp.dot(feat.astype(_BF16), wf2_ref[...],
                   preferred_element_type=_F32) + bf2_ref[...])
    out_ref[...] = out[:, :10]


def kernel(x, w1a, w1b, b1, w2a, w2b, b2, wf1, bf1, wf2, bf2):
    b = x.shape[0]
    xs = x.reshape(b, 28, 28)
    if b <= 1024:
        tb, pad = b, 0
    else:
        tb = 1024
        pad = (-b) % tb
    if pad:
        xs = jnp.pad(xs, ((0, pad), (0, 0), (0, 0)))

    packed = pl.pallas_call(
        _pack_body,
        out_shape=(jax.ShapeDtypeStruct((224, 1024), _BF16),
                   jax.ShapeDtypeStruct((1, 1536), _F32),
                   jax.ShapeDtypeStruct((768, 512), _BF16),
                   jax.ShapeDtypeStruct((1, 512), _F32),
                   jax.ShapeDtypeStruct((512, 128), _BF16),
                   jax.ShapeDtypeStruct((128, 128), _BF16)),
    )(w1a, w1b, b1, w2a, w2b, b2, wf1, wf2)
    w1p, b1t, w2p, b2t, wf1p, wf2p = packed

    bp = xs.shape[0]
    c2 = lambda i: (0, 0)
    feat, out = pl.pallas_call(
        _net_body,
        out_shape=(jax.ShapeDtypeStruct((bp, 128), _F32),
                   jax.ShapeDtypeStruct((bp, 10), _F32)),
        grid=(bp // tb,),
        in_specs=[
            pl.BlockSpec((tb, 28, 28), lambda i: (i, 0, 0)),
            pl.BlockSpec((224, 1024), c2),
            pl.BlockSpec((1, 1536), c2),
            pl.BlockSpec((768, 512), c2),
            pl.BlockSpec((1, 512), c2),
            pl.BlockSpec((512, 128), c2),
            pl.BlockSpec((1, 128), c2),
            pl.BlockSpec((128, 128), c2),
            pl.BlockSpec((1, 128), c2),
        ],
        out_specs=[
            pl.BlockSpec((tb, 128), lambda i: (i, 0)),
            pl.BlockSpec((tb, 10), lambda i: (i, 0)),
        ],
        compiler_params=pltpu.CompilerParams(
            dimension_semantics=("parallel",),
            vmem_limit_bytes=64 * 1024 * 1024,
        ),
    )(xs, w1p, b1t, w2p, b2t, wf1p, bf1, wf2p, bf2)
    if pad:
        feat, out = feat[:b], out[:b]
    return feat, out


# speedup vs baseline: 1.7866x; 1.0017x over previous
import jax
import jax.numpy as jnp
from jax.experimental import pallas as pl
from jax.experimental.pallas import tpu as pltpu

_BF16 = jnp.bfloat16
_F32 = jnp.float32


def _pack_values(w1a, w1b, b1, w2a, w2b, b2, wf1):
    s1 = [jnp.pad(w1a, ((0, 0), (0, 0), (0, 8))),
          jnp.pad(w1b, ((0, 0), (0, 0), (0, 8)))]
    w1 = jnp.concatenate(
        [jnp.pad(s, ((j, 3 - j), (0, 0), (0, 0))).reshape(224, 128)
         for j in range(4) for s in s1], axis=1)
    s2 = [jnp.pad(w2a, ((0, 0), (0, 8), (0, 48))),
          jnp.pad(w2b, ((0, 0), (0, 8), (0, 48)))]
    w2 = jnp.concatenate(
        [jnp.pad(s, ((j, 1 - j), (0, 0), (0, 0))).reshape(768, 128)
         for j in range(2) for s in s2], axis=1)
    wf1p = jnp.pad(wf1, ((0, 0), (0, 48), (0, 0))).reshape(512, 128)
    b1t = jnp.tile(jnp.pad(b1, ((0, 0), (0, 8))), (1, 12))
    b2t = jnp.tile(jnp.pad(b2, ((0, 0), (0, 48))), (1, 4))
    return w1.astype(_BF16), b1t, w2.astype(_BF16), b2t, wf1p.astype(_BF16)


def _net_body(x_ref, w1a_ref, w1b_ref, b1_ref, w2a_ref, w2b_ref, b2_ref,
              wf1_ref, bf1_ref, wf2_ref, bf2_ref, feat_ref, out_ref,
              w1_s, b1_s, w2_s, b2_s, wf1_s, wf2_s):
    tb = x_ref.shape[0]

    @pl.when(pl.program_id(0) == 0)
    def _pack():
        w1, b1t, w2, b2t, wf1p = _pack_values(
            w1a_ref[...], w1b_ref[...], b1_ref[...], w2a_ref[...],
            w2b_ref[...], b2_ref[...], wf1_ref[...])
        w1_s[...] = w1
        b1_s[...] = b1t
        w2_s[...] = w2
        b2_s[...] = b2t
        wf1_s[...] = wf1p
        wf2_s[...] = wf2_ref[...].astype(_BF16)

    xv = x_ref[...].astype(_BF16).reshape(tb, 784)

    hp_chunks = []
    for g in range(6):
        z = jnp.dot(xv[:, 112 * g:112 * g + 224], w1_s[...],
                    preferred_element_type=_F32)
        for j2 in range(2):
            s = 512 * j2
            hp_chunks.append(jnp.maximum(
                jnp.maximum(z[:, s:s + 128], z[:, s + 128:s + 256]),
                jnp.maximum(z[:, s + 256:s + 384], z[:, s + 384:s + 512])))
    p1 = jnp.concatenate(hp_chunks, axis=1)
    p1 = jnp.maximum(p1 + b1_s[...], 0.0).astype(_BF16)

    hp2_chunks = []
    for g in range(4):
        z = jnp.dot(p1[:, 256 * g:256 * g + 768], w2_s[...],
                    preferred_element_type=_F32)
        hp2_chunks.append(jnp.maximum(
            jnp.maximum(z[:, 0:128], z[:, 128:256]),
            jnp.maximum(z[:, 256:384], z[:, 384:512])))
    p2 = jnp.concatenate(hp2_chunks, axis=1)
    p2 = jnp.maximum(p2 + b2_s[...], 0.0).astype(_BF16)

    feat = jnp.maximum(
        jnp.dot(p2, wf1_s[...], preferred_element_type=_F32) + bf1_ref[...],
        0.0)
    feat_ref[...] = feat
    out = (jnp.dot(feat.astype(_BF16), wf2_s[...],
                   preferred_element_type=_F32) + bf2_ref[...])
    out_ref[...] = out[:, :10]


def kernel(x, w1a, w1b, b1, w2a, w2b, b2, wf1, bf1, wf2, bf2):
    b = x.shape[0]
    xs = x.reshape(b, 28, 28)
    if b <= 1024:
        tb, pad = b, 0
    else:
        tb = 1024
        pad = (-b) % tb
    if pad:
        xs = jnp.pad(xs, ((0, pad), (0, 0), (0, 0)))

    bp = xs.shape[0]
    c2 = lambda i: (0, 0)
    c3 = lambda i: (0, 0, 0)
    feat, out = pl.pallas_call(
        _net_body,
        out_shape=(jax.ShapeDtypeStruct((bp, 128), _F32),
                   jax.ShapeDtypeStruct((bp, 10), _F32)),
        grid=(bp // tb,),
        in_specs=[
            pl.BlockSpec((tb, 28, 28), lambda i: (i, 0, 0)),
            pl.BlockSpec((5, 28, 120), c3),
            pl.BlockSpec((5, 28, 120), c3),
            pl.BlockSpec((1, 120), c2),
            pl.BlockSpec((5, 120, 80), c3),
            pl.BlockSpec((5, 120, 80), c3),
            pl.BlockSpec((1, 80), c2),
            pl.BlockSpec((4, 80, 128), c3),
            pl.BlockSpec((1, 128), c2),
            pl.BlockSpec((128, 128), c2),
            pl.BlockSpec((1, 128), c2),
        ],
        out_specs=[
            pl.BlockSpec((tb, 128), lambda i: (i, 0)),
            pl.BlockSpec((tb, 10), lambda i: (i, 0)),
        ],
        scratch_shapes=[
            pltpu.VMEM((224, 1024), _BF16),
            pltpu.VMEM((1, 1536), _F32),
            pltpu.VMEM((768, 512), _BF16),
            pltpu.VMEM((1, 512), _F32),
            pltpu.VMEM((512, 128), _BF16),
            pltpu.VMEM((128, 128), _BF16),
        ],
        compiler_params=pltpu.CompilerParams(
            dimension_semantics=("arbitrary",),
            vmem_limit_bytes=64 * 1024 * 1024,
        ),
    )(xs, w1a, w1b, b1, w2a, w2b, b2, wf1, bf1, wf2, bf2)
    if pad:
        feat, out = feat[:b], out[:b]
    return feat, out
